# Initial kernel scaffold; baseline (speedup 1.0000x reference)
import jax
import jax.numpy as jnp
from jax.experimental import pallas as pl


def kernel(x, dutyCycle):
    raise NotImplementedError("write your pallas kernel here")


# speedup vs baseline: 26.5223x; 26.5223x over previous
import jax
import jax.numpy as jnp
from jax.experimental import pallas as pl
from jax.experimental.pallas import tpu as pltpu

_B = 32
_C = 192
_H = 56
_W = 56
_N = _C * _H * _W
_K = int(round(_N * 0.1))
_LANES = 128
_ROWS = _N // _LANES
_BOOST_STRENGTH = 1.0


def _body(x_ref, bf_ref, out_ref, keys_ref):
    x = x_ref[0]
    boosted = x * bf_ref[...]
    i = jax.lax.bitcast_convert_type(boosted, jnp.int32)
    keys_ref[...] = i ^ ((i >> 31) & jnp.int32(0x7FFFFFFF))

    def step(_, carry):
        lo, hi = carry
        mid = (lo | hi) - ((lo ^ hi) >> 1)
        cnt = jnp.sum(jnp.where(keys_ref[...] >= mid, jnp.int32(1), jnp.int32(0)))
        ok = cnt >= jnp.int32(_K)
        return (jnp.where(ok, mid, lo), jnp.where(ok, hi, mid - jnp.int32(1)))

    lo0 = jnp.int32(-(2 ** 31))
    hi0 = jnp.int32(2 ** 31 - 1)
    thresh, _ = jax.lax.fori_loop(0, 32, step, (lo0, hi0))
    out_ref[0] = jnp.where(keys_ref[...] >= thresh, x, jnp.float32(0.0))


def kernel(x, dutyCycle):
    target_density = jnp.float32(float(_K) / float(_N))
    bf = jnp.exp((target_density - dutyCycle.reshape(_C)) * jnp.float32(_BOOST_STRENGTH))
    bf_full = jnp.repeat(bf, _H * _W).reshape(_ROWS, _LANES)
    xr = x.reshape(_B, _ROWS, _LANES)
    out = pl.pallas_call(
        _body,
        grid=(_B,),
        in_specs=[
            pl.BlockSpec((1, _ROWS, _LANES), lambda b: (b, 0, 0)),
            pl.BlockSpec((_ROWS, _LANES), lambda b: (0, 0)),
        ],
        out_specs=pl.BlockSpec((1, _ROWS, _LANES), lambda b: (b, 0, 0)),
        out_shape=jax.ShapeDtypeStruct((_B, _ROWS, _LANES), jnp.float32),
        scratch_shapes=[pltpu.VMEM((_ROWS, _LANES), jnp.int32)],
    )(xr, bf_full)
    return out.reshape(_B, _C, _H, _W)


# speedup vs baseline: 37.0300x; 1.3962x over previous
import jax
import jax.numpy as jnp
from jax.experimental import pallas as pl
from jax.experimental.pallas import tpu as pltpu

_B = 32
_C = 192
_H = 56
_W = 56
_N = _C * _H * _W
_K = int(round(_N * 0.1))
_LANES = 128
_ROWS = _N // _LANES
_BOOST_STRENGTH = 1.0


_NCHUNK = 12
_VREGS_PER_CHUNK = _ROWS // 8 // _NCHUNK


def _body(x_ref, bf_ref, out_ref, keys_ref):
    x = x_ref[0]
    boosted = x * bf_ref[...]
    i = jax.lax.bitcast_convert_type(boosted, jnp.int32)
    key = i ^ ((i >> 31) & jnp.int32(0x7FFFFFFF))
    keys_ref[...] = key.reshape(_NCHUNK, _VREGS_PER_CHUNK, 8, _LANES)

    def step(_, carry):
        lo, hi = carry
        mid = (lo | hi) - ((lo ^ hi) >> 1)
        acc = None
        for g in range(_NCHUNK):
            m = jnp.where(keys_ref[g] >= mid, jnp.float32(1.0), jnp.float32(0.0))
            s = jnp.sum(m, axis=0)
            acc = s if acc is None else acc + s
        cnt = jnp.sum(acc)
        ok = cnt >= jnp.float32(_K)
        return (jnp.where(ok, mid, lo), jnp.where(ok, hi, mid - jnp.int32(1)))

    lo0 = jnp.int32(-(2 ** 31))
    hi0 = jnp.int32(2 ** 31 - 1)
    thresh, _ = jax.lax.fori_loop(0, 32, step, (lo0, hi0))
    keys2d = keys_ref[...].reshape(_ROWS, _LANES)
    out_ref[0] = jnp.where(keys2d >= thresh, x, jnp.float32(0.0))


def kernel(x, dutyCycle):
    target_density = jnp.float32(float(_K) / float(_N))
    bf = jnp.exp((target_density - dutyCycle.reshape(_C)) * jnp.float32(_BOOST_STRENGTH))
    bf_full = jnp.repeat(bf, _H * _W).reshape(_ROWS, _LANES)
    xr = x.reshape(_B, _ROWS, _LANES)
    out = pl.pallas_call(
        _body,
        grid=(_B,),
        in_specs=[
            pl.BlockSpec((1, _ROWS, _LANES), lambda b: (b, 0, 0)),
            pl.BlockSpec((_ROWS, _LANES), lambda b: (0, 0)),
        ],
        out_specs=pl.BlockSpec((1, _ROWS, _LANES), lambda b: (b, 0, 0)),
        out_shape=jax.ShapeDtypeStruct((_B, _ROWS, _LANES), jnp.float32),
        scratch_shapes=[pltpu.VMEM((_NCHUNK, _VREGS_PER_CHUNK, 8, _LANES), jnp.int32)],
    )(xr, bf_full)
    return out.reshape(_B, _C, _H, _W)


# speedup vs baseline: 39.5734x; 1.0687x over previous
import jax
import jax.numpy as jnp
from jax.experimental import pallas as pl
from jax.experimental.pallas import tpu as pltpu

_B = 32
_C = 192
_H = 56
_W = 56
_N = _C * _H * _W
_K = int(round(_N * 0.1))
_LANES = 128
_ROWS = _N // _LANES
_BOOST_STRENGTH = 1.0
_S = 2
_NCHUNK = 12
_VREGS_PER_CHUNK = _ROWS // 8 // _NCHUNK


def _body(x_ref, bf_ref, out_ref, keys_ref):
    bf = bf_ref[...]
    for s in range(_S):
        boosted = x_ref[s] * bf
        i = jax.lax.bitcast_convert_type(boosted, jnp.int32)
        key = i ^ ((i >> 31) & jnp.int32(0x7FFFFFFF))
        keys_ref[s] = key.reshape(_NCHUNK, _VREGS_PER_CHUNK, 8, _LANES)

    def count_ge(s, mid):
        acc = None
        for g in range(_NCHUNK):
            m = jnp.where(keys_ref[s, g] >= mid, jnp.float32(1.0), jnp.float32(0.0))
            ps = jnp.sum(m, axis=0)
            acc = ps if acc is None else acc + ps
        return jnp.sum(acc)

    def step(_, carry):
        nxt = []
        for s in range(_S):
            lo, hi = carry[2 * s], carry[2 * s + 1]
            mid = (lo | hi) - ((lo ^ hi) >> 1)
            ok = count_ge(s, mid) >= jnp.float32(_K)
            nxt.append(jnp.where(ok, mid, lo))
            nxt.append(jnp.where(ok, hi, mid - jnp.int32(1)))
        return tuple(nxt)

    lo0 = jnp.int32(-(2 ** 31))
    hi0 = jnp.int32(2 ** 31 - 1)
    res = jax.lax.fori_loop(0, 32, step, (lo0, hi0) * _S)
    for s in range(_S):
        keys2d = keys_ref[s].reshape(_ROWS, _LANES)
        out_ref[s] = jnp.where(keys2d >= res[2 * s], x_ref[s], jnp.float32(0.0))


def kernel(x, dutyCycle):
    target_density = jnp.float32(float(_K) / float(_N))
    bf = jnp.exp((target_density - dutyCycle.reshape(_C)) * jnp.float32(_BOOST_STRENGTH))
    bf_full = jnp.repeat(bf, _H * _W).reshape(_ROWS, _LANES)
    xr = x.reshape(_B, _ROWS, _LANES)
    out = pl.pallas_call(
        _body,
        grid=(_B // _S,),
        in_specs=[
            pl.BlockSpec((_S, _ROWS, _LANES), lambda b: (b, 0, 0)),
            pl.BlockSpec((_ROWS, _LANES), lambda b: (0, 0)),
        ],
        out_specs=pl.BlockSpec((_S, _ROWS, _LANES), lambda b: (b, 0, 0)),
        out_shape=jax.ShapeDtypeStruct((_B, _ROWS, _LANES), jnp.float32),
        scratch_shapes=[
            pltpu.VMEM((_S, _NCHUNK, _VREGS_PER_CHUNK, 8, _LANES), jnp.int32)
        ],
    )(xr, bf_full)
    return out.reshape(_B, _C, _H, _W)
